# Initial kernel scaffold; baseline (speedup 1.0000x reference)
import jax
import jax.numpy as jnp
from jax.experimental import pallas as pl


def kernel(x, Wr, br, We, be):
    raise NotImplementedError("write your pallas kernel here")


# speedup vs baseline: 1.7361x; 1.7361x over previous
import functools

import jax
import jax.numpy as jnp
from jax import lax
from jax.experimental import pallas as pl
from jax.experimental.pallas import tpu as pltpu
from jax.experimental.pallas import tpu_sc as plsc

E = 16
D = 2048
H = 2048
T = 4096

M = 128
G = T // M + E
GP = G * M
HB = 1024
TB = 512

_NC = 2
_NS = 16
_NW = _NC * _NS


def _router_body(x_ref, wr_ref, br_ref, probs_ref, counts_ref):
    i = pl.program_id(0)
    logits = jnp.dot(x_ref[...], wr_ref[...],
                     preferred_element_type=jnp.float32) + br_ref[...]
    m = jnp.max(logits, axis=-1, keepdims=True)
    ex = jnp.exp(logits - m)
    s = jnp.sum(ex, axis=-1, keepdims=True)
    probs = ex / s
    probs_ref[...] = probs
    routes = jnp.argmax(probs, axis=-1).astype(jnp.int32)
    iota = lax.broadcasted_iota(jnp.int32, (TB, 128), 1)
    onehot = (iota == routes[:, None]).astype(jnp.float32)
    hist = jnp.sum(onehot, axis=0, keepdims=True)

    @pl.when(i == 0)
    def _():
        counts_ref[...] = jnp.zeros_like(counts_ref)

    counts_ref[0:1, :] = counts_ref[0:1, :] + hist


def _router(x, Wr, br):
    return pl.pallas_call(
        _router_body,
        grid=(T // TB,),
        in_specs=[
            pl.BlockSpec((TB, D), lambda i: (i, 0)),
            pl.BlockSpec((D, E), lambda i: (0, 0)),
            pl.BlockSpec((1, E), lambda i: (0, 0)),
        ],
        out_specs=[
            pl.BlockSpec((TB, E), lambda i: (i, 0)),
            pl.BlockSpec((8, 128), lambda i: (0, 0)),
        ],
        out_shape=[
            jax.ShapeDtypeStruct((T, E), jnp.float32),
            jax.ShapeDtypeStruct((8, 128), jnp.float32),
        ],
    )(x, Wr, br.reshape(1, E))


def _gather_rows(table, idx):
    B = idx.shape[0]
    Dd = table.shape[1]
    b_per_w = B // _NW
    CH = 32
    nch = b_per_w // CH
    mesh = plsc.VectorSubcoreMesh(core_axis_name="c", subcore_axis_name="s")

    @functools.partial(
        pl.kernel,
        out_type=jax.ShapeDtypeStruct((B, Dd), jnp.float32),
        mesh=mesh,
        scratch_types=[
            pltpu.VMEM((CH,), jnp.int32),
            pltpu.VMEM((CH, Dd), jnp.float32),
            pltpu.SemaphoreType.DMA,
        ],
    )
    def k(table_hbm, idx_hbm, out_hbm, idx_v, rows_v, sem):
        wid = lax.axis_index("s") * _NC + lax.axis_index("c")
        base = wid * b_per_w
        for j in range(nch):
            pltpu.sync_copy(idx_hbm.at[pl.ds(base + j * CH, CH)], idx_v)
            pltpu.async_copy(table_hbm.at[idx_v], rows_v, sem).wait()
            pltpu.sync_copy(rows_v, out_hbm.at[pl.ds(base + j * CH, CH)])

    return k(table, idx)


def _mm_body(te_ref, xs_ref, we_ref, be_ref, out_ref):
    out_ref[...] = jnp.dot(xs_ref[...], we_ref[0],
                           preferred_element_type=jnp.float32) + be_ref[0]


def _grouped_matmul(te, xs, We, be):
    grid_spec = pltpu.PrefetchScalarGridSpec(
        num_scalar_prefetch=1,
        grid=(H // HB, G),
        in_specs=[
            pl.BlockSpec((M, D), lambda h, g, te_ref: (g, 0)),
            pl.BlockSpec((1, D, HB), lambda h, g, te_ref: (te_ref[g], 0, h)),
            pl.BlockSpec((1, 1, HB), lambda h, g, te_ref: (te_ref[g], 0, h)),
        ],
        out_specs=pl.BlockSpec((M, HB), lambda h, g, te_ref: (g, h)),
    )
    return pl.pallas_call(
        _mm_body,
        grid_spec=grid_spec,
        out_shape=jax.ShapeDtypeStruct((GP, H), jnp.float32),
    )(te, xs, We, be.reshape(E, 1, H))


def kernel(x, Wr, br, We, be):
    probs, counts8 = _router(x, Wr, br)
    counts = counts8[0, :E]

    i32 = jnp.int32
    routes = jnp.argmax(probs, axis=-1).astype(i32)
    counts_i = counts.astype(i32)
    order = jnp.argsort(routes, stable=True).astype(i32)
    offs = jnp.concatenate([jnp.zeros(1, i32),
                            jnp.cumsum(counts_i)[:-1].astype(i32)])
    padded_sizes = ((counts_i + M - 1) // M) * M
    poffs = jnp.concatenate([jnp.zeros(1, i32),
                             jnp.cumsum(padded_sizes)[:-1].astype(i32)])

    rank = jnp.zeros(T, i32).at[order].set(jnp.arange(T, dtype=i32))
    pos = poffs[routes] + (rank - offs[routes])

    r = jnp.arange(GP, dtype=i32)
    e_r = (jnp.searchsorted(poffs, r, side="right").astype(i32) - 1)
    i_in = r - poffs[e_r]
    valid = i_in < counts_i[e_r]
    src = offs[e_r] + jnp.minimum(i_in, jnp.maximum(counts_i[e_r] - 1, 0))
    src = jnp.clip(src, 0, T - 1)
    tok = jnp.where(valid, order[src], 0).astype(i32)
    te = e_r[::M]

    xs = _gather_rows(x, tok)
    ys = _grouped_matmul(te, xs, We, be)
    out = _gather_rows(ys, pos)

    return out, probs, counts


# speedup vs baseline: 2.2576x; 1.3004x over previous
import functools

import jax
import jax.numpy as jnp
from jax import lax
from jax.experimental import pallas as pl
from jax.experimental.pallas import tpu as pltpu
from jax.experimental.pallas import tpu_sc as plsc

E = 16
D = 2048
H = 2048
T = 4096

M = 128
G = T // M + E
GP = G * M
HB = 1024
TB = 512

_NC = 2
_NS = 16
_NW = _NC * _NS


def _router_body(x_ref, wr_ref, br_ref, probs_ref, counts_ref):
    i = pl.program_id(0)
    logits = jnp.dot(x_ref[...], wr_ref[...],
                     preferred_element_type=jnp.float32) + br_ref[...]
    m = jnp.max(logits, axis=-1, keepdims=True)
    ex = jnp.exp(logits - m)
    s = jnp.sum(ex, axis=-1, keepdims=True)
    probs = ex / s
    probs_ref[...] = probs
    routes = jnp.argmax(probs, axis=-1).astype(jnp.int32)
    iota = lax.broadcasted_iota(jnp.int32, (TB, 128), 1)
    onehot = (iota == routes[:, None]).astype(jnp.float32)
    hist = jnp.sum(onehot, axis=0, keepdims=True)

    @pl.when(i == 0)
    def _():
        counts_ref[...] = jnp.zeros_like(counts_ref)

    counts_ref[0:1, :] = counts_ref[0:1, :] + hist


def _router(x, Wr, br):
    return pl.pallas_call(
        _router_body,
        grid=(T // TB,),
        in_specs=[
            pl.BlockSpec((TB, D), lambda i: (i, 0)),
            pl.BlockSpec((D, E), lambda i: (0, 0)),
            pl.BlockSpec((1, E), lambda i: (0, 0)),
        ],
        out_specs=[
            pl.BlockSpec((TB, E), lambda i: (i, 0)),
            pl.BlockSpec((8, 128), lambda i: (0, 0)),
        ],
        out_shape=[
            jax.ShapeDtypeStruct((T, E), jnp.float32),
            jax.ShapeDtypeStruct((8, 128), jnp.float32),
        ],
    )(x, Wr, br.reshape(1, E))


def _gather_rows(table, idx):
    B = idx.shape[0]
    Dd = table.shape[1]
    b_per_w = B // _NW
    CH = 32
    nch = b_per_w // CH
    mesh = plsc.VectorSubcoreMesh(core_axis_name="c", subcore_axis_name="s")

    @functools.partial(
        pl.kernel,
        out_type=jax.ShapeDtypeStruct((B, Dd), jnp.float32),
        mesh=mesh,
        scratch_types=[
            pltpu.VMEM((CH,), jnp.int32),
            pltpu.VMEM((CH, Dd), jnp.float32),
            pltpu.SemaphoreType.DMA,
        ],
    )
    def k(table_hbm, idx_hbm, out_hbm, idx_v, rows_v, sem):
        wid = lax.axis_index("s") * _NC + lax.axis_index("c")
        base = wid * b_per_w
        for j in range(nch):
            pltpu.sync_copy(idx_hbm.at[pl.ds(base + j * CH, CH)], idx_v)
            pltpu.async_copy(table_hbm.at[idx_v], rows_v, sem).wait()
            pltpu.sync_copy(rows_v, out_hbm.at[pl.ds(base + j * CH, CH)])

    return k(table, idx)


def _mm_body(te_ref, xs_ref, we_ref, be_ref, out_ref):
    out_ref[...] = jnp.dot(xs_ref[...].astype(jnp.bfloat16),
                           we_ref[0].astype(jnp.bfloat16),
                           preferred_element_type=jnp.float32) + be_ref[0]


def _grouped_matmul(te, xs, We, be):
    grid_spec = pltpu.PrefetchScalarGridSpec(
        num_scalar_prefetch=1,
        grid=(H // HB, G),
        in_specs=[
            pl.BlockSpec((M, D), lambda h, g, te_ref: (g, 0)),
            pl.BlockSpec((1, D, HB), lambda h, g, te_ref: (te_ref[g], 0, h)),
            pl.BlockSpec((1, 1, HB), lambda h, g, te_ref: (te_ref[g], 0, h)),
        ],
        out_specs=pl.BlockSpec((M, HB), lambda h, g, te_ref: (g, h)),
    )
    return pl.pallas_call(
        _mm_body,
        grid_spec=grid_spec,
        out_shape=jax.ShapeDtypeStruct((GP, H), jnp.float32),
    )(te, xs, We, be.reshape(E, 1, H))


def kernel(x, Wr, br, We, be):
    probs, counts8 = _router(x, Wr, br)
    counts = counts8[0, :E]

    i32 = jnp.int32
    routes = jnp.argmax(probs, axis=-1).astype(i32)
    counts_i = counts.astype(i32)
    order = jnp.argsort(routes, stable=True).astype(i32)
    offs = jnp.concatenate([jnp.zeros(1, i32),
                            jnp.cumsum(counts_i)[:-1].astype(i32)])
    padded_sizes = ((counts_i + M - 1) // M) * M
    poffs = jnp.concatenate([jnp.zeros(1, i32),
                             jnp.cumsum(padded_sizes)[:-1].astype(i32)])

    rank = jnp.zeros(T, i32).at[order].set(jnp.arange(T, dtype=i32))
    pos = poffs[routes] + (rank - offs[routes])

    r = jnp.arange(GP, dtype=i32)
    e_r = (jnp.searchsorted(poffs, r, side="right").astype(i32) - 1)
    i_in = r - poffs[e_r]
    valid = i_in < counts_i[e_r]
    src = offs[e_r] + jnp.minimum(i_in, jnp.maximum(counts_i[e_r] - 1, 0))
    src = jnp.clip(src, 0, T - 1)
    tok = jnp.where(valid, order[src], r % T).astype(i32)
    te = e_r[::M]

    xs = _gather_rows(x, tok)
    ys = _grouped_matmul(te, xs, We, be)
    out = _gather_rows(ys, pos)

    return out, probs, counts


# speedup vs baseline: 3.0821x; 1.3652x over previous
import functools

import jax
import jax.numpy as jnp
from jax import lax
from jax.experimental import pallas as pl
from jax.experimental.pallas import tpu as pltpu
from jax.experimental.pallas import tpu_sc as plsc

E = 16
D = 2048
H = 2048
T = 4096

M = 128
G = T // M + E
GP = G * M
HB = 1024
TB = 512

_NC = 2
_NS = 16
_NW = _NC * _NS


def _router_body(x_ref, wr_ref, br_ref, probs_ref, counts_ref, cum_ref):
    i = pl.program_id(0)
    logits = jnp.dot(x_ref[...], wr_ref[...],
                     preferred_element_type=jnp.float32) + br_ref[...]
    m = jnp.max(logits, axis=-1, keepdims=True)
    ex = jnp.exp(logits - m)
    s = jnp.sum(ex, axis=-1, keepdims=True)
    probs = ex / s
    probs_ref[...] = probs
    routes = jnp.argmax(probs, axis=-1).astype(jnp.int32)
    iota = lax.broadcasted_iota(jnp.int32, (TB, 128), 1)
    onehot = (iota == routes[:, None]).astype(jnp.bfloat16)
    hist = jnp.sum(onehot.astype(jnp.float32), axis=0, keepdims=True)

    @pl.when(i == 0)
    def _():
        counts_ref[...] = jnp.zeros_like(counts_ref)

    carry = counts_ref[0:1, :]
    r_i = lax.broadcasted_iota(jnp.int32, (TB, TB), 0)
    c_i = lax.broadcasted_iota(jnp.int32, (TB, TB), 1)
    tri = (c_i < r_i).astype(jnp.bfloat16)
    within = jnp.dot(tri, onehot, preferred_element_type=jnp.float32)
    cum_ref[...] = within[:, :E] + carry[:, :E]
    counts_ref[0:1, :] = carry + hist


def _router(x, Wr, br):
    return pl.pallas_call(
        _router_body,
        grid=(T // TB,),
        in_specs=[
            pl.BlockSpec((TB, D), lambda i: (i, 0)),
            pl.BlockSpec((D, E), lambda i: (0, 0)),
            pl.BlockSpec((1, E), lambda i: (0, 0)),
        ],
        out_specs=[
            pl.BlockSpec((TB, E), lambda i: (i, 0)),
            pl.BlockSpec((8, 128), lambda i: (0, 0)),
            pl.BlockSpec((TB, E), lambda i: (i, 0)),
        ],
        out_shape=[
            jax.ShapeDtypeStruct((T, E), jnp.float32),
            jax.ShapeDtypeStruct((8, 128), jnp.float32),
            jax.ShapeDtypeStruct((T, E), jnp.float32),
        ],
    )(x, Wr, br.reshape(1, E))


def _gather_rows(table, idx):
    B = idx.shape[0]
    Dd = table.shape[1]
    b_per_w = B // _NW
    CH = 32
    nch = b_per_w // CH
    mesh = plsc.VectorSubcoreMesh(core_axis_name="c", subcore_axis_name="s")

    @functools.partial(
        pl.kernel,
        out_type=jax.ShapeDtypeStruct((B, Dd), jnp.float32),
        mesh=mesh,
        scratch_types=[
            pltpu.VMEM((CH,), jnp.int32),
            pltpu.VMEM((CH, Dd), jnp.float32),
            pltpu.SemaphoreType.DMA,
        ],
    )
    def k(table_hbm, idx_hbm, out_hbm, idx_v, rows_v, sem):
        wid = lax.axis_index("s") * _NC + lax.axis_index("c")
        base = wid * b_per_w
        for j in range(nch):
            pltpu.sync_copy(idx_hbm.at[pl.ds(base + j * CH, CH)], idx_v)
            pltpu.async_copy(table_hbm.at[idx_v], rows_v, sem).wait()
            pltpu.sync_copy(rows_v, out_hbm.at[pl.ds(base + j * CH, CH)])

    return k(table, idx)


def _mm_body(te_ref, xs_ref, we_ref, be_ref, out_ref):
    out_ref[...] = jnp.dot(xs_ref[...].astype(jnp.bfloat16),
                           we_ref[0].astype(jnp.bfloat16),
                           preferred_element_type=jnp.float32) + be_ref[0]


def _grouped_matmul(te, xs, We, be):
    grid_spec = pltpu.PrefetchScalarGridSpec(
        num_scalar_prefetch=1,
        grid=(G,),
        in_specs=[
            pl.BlockSpec((M, D), lambda g, te_ref: (g, 0)),
            pl.BlockSpec((1, D, H), lambda g, te_ref: (te_ref[g], 0, 0)),
            pl.BlockSpec((1, 1, H), lambda g, te_ref: (te_ref[g], 0, 0)),
        ],
        out_specs=pl.BlockSpec((M, H), lambda g, te_ref: (g, 0)),
    )
    return pl.pallas_call(
        _mm_body,
        grid_spec=grid_spec,
        out_shape=jax.ShapeDtypeStruct((GP, H), jnp.float32),
    )(te, xs, We, be.reshape(E, 1, H))


def kernel(x, Wr, br, We, be):
    probs, counts8, cum = _router(x, Wr, br)
    counts = counts8[0, :E]

    i32 = jnp.int32
    routes = jnp.argmax(probs, axis=-1).astype(i32)
    rank = jnp.take_along_axis(cum, routes[:, None], axis=1)[:, 0].astype(i32)
    counts_i = counts.astype(i32)
    padded_sizes = ((counts_i + M - 1) // M) * M
    poffs = jnp.concatenate([jnp.zeros(1, i32),
                             jnp.cumsum(padded_sizes)[:-1].astype(i32)])

    pos = poffs[routes] + rank
    r = jnp.arange(GP, dtype=i32)
    tok = (r % T).at[pos].set(jnp.arange(T, dtype=i32))
    te = (jnp.searchsorted(poffs, jnp.arange(G, dtype=i32) * M,
                           side="right").astype(i32) - 1)

    xs = _gather_rows(x, tok)
    ys = _grouped_matmul(te, xs, We, be)
    out = _gather_rows(ys, pos)

    return out, probs, counts


# speedup vs baseline: 3.4168x; 1.1086x over previous
import functools

import jax
import jax.numpy as jnp
from jax import lax
from jax.experimental import pallas as pl
from jax.experimental.pallas import tpu as pltpu
from jax.experimental.pallas import tpu_sc as plsc

E = 16
D = 2048
H = 2048
T = 4096

M = 128
G = T // M + E
GP = G * M
HB = 1024
TB = 512

_NC = 2
_NS = 16
_NW = _NC * _NS


def _router_body(x_ref, wr_ref, br_ref, probs_ref, counts_ref, cum_ref):
    i = pl.program_id(0)
    logits = jnp.dot(x_ref[...], wr_ref[...],
                     preferred_element_type=jnp.float32) + br_ref[...]
    m = jnp.max(logits, axis=-1, keepdims=True)
    ex = jnp.exp(logits - m)
    s = jnp.sum(ex, axis=-1, keepdims=True)
    probs = ex / s
    probs_ref[...] = probs
    routes = jnp.argmax(probs, axis=-1).astype(jnp.int32)
    iota = lax.broadcasted_iota(jnp.int32, (TB, 128), 1)
    onehot = (iota == routes[:, None]).astype(jnp.bfloat16)
    hist = jnp.sum(onehot.astype(jnp.float32), axis=0, keepdims=True)

    @pl.when(i == 0)
    def _():
        counts_ref[...] = jnp.zeros_like(counts_ref)

    carry = counts_ref[0:1, :]
    r_i = lax.broadcasted_iota(jnp.int32, (TB, TB), 0)
    c_i = lax.broadcasted_iota(jnp.int32, (TB, TB), 1)
    tri = (c_i < r_i).astype(jnp.bfloat16)
    within = jnp.dot(tri, onehot, preferred_element_type=jnp.float32)
    cum_ref[...] = within[:, :E] + carry[:, :E]
    counts_ref[0:1, :] = carry + hist


def _router(x, Wr, br):
    return pl.pallas_call(
        _router_body,
        grid=(T // TB,),
        in_specs=[
            pl.BlockSpec((TB, D), lambda i: (i, 0)),
            pl.BlockSpec((D, E), lambda i: (0, 0)),
            pl.BlockSpec((1, E), lambda i: (0, 0)),
        ],
        out_specs=[
            pl.BlockSpec((TB, E), lambda i: (i, 0)),
            pl.BlockSpec((8, 128), lambda i: (0, 0)),
            pl.BlockSpec((TB, E), lambda i: (i, 0)),
        ],
        out_shape=[
            jax.ShapeDtypeStruct((T, E), jnp.float32),
            jax.ShapeDtypeStruct((8, 128), jnp.float32),
            jax.ShapeDtypeStruct((T, E), jnp.float32),
        ],
    )(x, Wr, br.reshape(1, E))


def _gather_rows(table, idx):
    B = idx.shape[0]
    Dd = table.shape[1]
    b_per_w = B // _NW
    CH = 32
    nch = b_per_w // CH
    mesh = plsc.VectorSubcoreMesh(core_axis_name="c", subcore_axis_name="s")

    @functools.partial(
        pl.kernel,
        out_type=jax.ShapeDtypeStruct((B, Dd), jnp.float32),
        mesh=mesh,
        scratch_types=[
            pltpu.VMEM((CH,), jnp.int32),
            pltpu.VMEM((CH, Dd), jnp.float32),
            pltpu.SemaphoreType.DMA,
        ],
    )
    def k(table_hbm, idx_hbm, out_hbm, idx_v, rows_v, sem):
        wid = lax.axis_index("s") * _NC + lax.axis_index("c")
        base = wid * b_per_w
        for j in range(nch):
            pltpu.sync_copy(idx_hbm.at[pl.ds(base + j * CH, CH)], idx_v)
            pltpu.async_copy(table_hbm.at[idx_v], rows_v, sem).wait()
            pltpu.sync_copy(rows_v, out_hbm.at[pl.ds(base + j * CH, CH)])

    return k(table, idx)


def _scatter_rows(src, pos, n_out):
    B, Dd = src.shape
    b_per_w = B // _NW
    CH = 32
    nch = b_per_w // CH
    mesh = plsc.VectorSubcoreMesh(core_axis_name="c", subcore_axis_name="s")

    @functools.partial(
        pl.kernel,
        out_type=jax.ShapeDtypeStruct((n_out, Dd), jnp.float32),
        mesh=mesh,
        scratch_types=[
            pltpu.VMEM((nch, CH), jnp.int32),
            pltpu.VMEM((CH, Dd), jnp.float32),
            pltpu.SemaphoreType.DMA,
        ],
    )
    def k(src_hbm, pos_hbm, out_hbm, idx_v, rows_v, sem):
        wid = lax.axis_index("s") * _NC + lax.axis_index("c")
        base = wid * b_per_w
        pltpu.sync_copy(pos_hbm.at[wid], idx_v)
        for j in range(nch):
            pltpu.sync_copy(src_hbm.at[pl.ds(base + j * CH, CH)], rows_v)
            pltpu.async_copy(rows_v, out_hbm.at[idx_v.at[j]], sem).wait()

    return k(src, pos.reshape(_NW, nch, CH))


def _mm_body(te_ref, xs_ref, we_ref, be_ref, out_ref):
    out_ref[...] = jnp.dot(xs_ref[...].astype(jnp.bfloat16),
                           we_ref[0].astype(jnp.bfloat16),
                           preferred_element_type=jnp.float32) + be_ref[0]


def _grouped_matmul(te, xs, We, be):
    grid_spec = pltpu.PrefetchScalarGridSpec(
        num_scalar_prefetch=1,
        grid=(G,),
        in_specs=[
            pl.BlockSpec((M, D), lambda g, te_ref: (g, 0)),
            pl.BlockSpec((1, D, H), lambda g, te_ref: (te_ref[g], 0, 0)),
            pl.BlockSpec((1, 1, H), lambda g, te_ref: (te_ref[g], 0, 0)),
        ],
        out_specs=pl.BlockSpec((M, H), lambda g, te_ref: (g, 0)),
    )
    return pl.pallas_call(
        _mm_body,
        grid_spec=grid_spec,
        out_shape=jax.ShapeDtypeStruct((GP, H), jnp.float32),
    )(te, xs, We, be.reshape(E, 1, H))


def kernel(x, Wr, br, We, be):
    probs, counts8, cum = _router(x, Wr, br)
    counts = counts8[0, :E]

    i32 = jnp.int32
    routes = jnp.argmax(probs, axis=-1).astype(i32)
    rank = jnp.take_along_axis(cum, routes[:, None], axis=1)[:, 0].astype(i32)
    counts_i = counts.astype(i32)
    padded_sizes = ((counts_i + M - 1) // M) * M
    poffs = jnp.concatenate([jnp.zeros(1, i32),
                             jnp.cumsum(padded_sizes)[:-1].astype(i32)])

    pos = poffs[routes] + rank
    te = (jnp.searchsorted(poffs, jnp.arange(G, dtype=i32) * M,
                           side="right").astype(i32) - 1)

    xs = _scatter_rows(x, pos, GP)
    ys = _grouped_matmul(te, xs, We, be)
    out = _gather_rows(ys, pos)

    return out, probs, counts
